# Initial kernel scaffold; baseline (speedup 1.0000x reference)
import jax
import jax.numpy as jnp
from jax.experimental import pallas as pl


def kernel(pos, vel_history, particle_type, edge_index, params):
    raise NotImplementedError("write your pallas kernel here")


# speedup vs baseline: 2.3080x; 2.3080x over previous
import functools

import jax
import jax.numpy as jnp
from jax import lax
from jax.experimental import pallas as pl
from jax.experimental.pallas import tpu as pltpu
from jax.experimental.pallas import tpu_sc as plsc

_N = 10000
_E = 320000
_H = 128
_NP = 10240
_BLK_N = 512
_NB_N = _NP // _BLK_N
_BLK_E = 512
_NB_E = _E // _BLK_E

_NC = 2
_NS = 16
_NW = _NC * _NS
_EPW = _E // _NW
_C = 80
_NCH = _EPW // _C
_RPT = _NP // _NS

_F32 = jnp.float32


def _sc_mesh():
    return plsc.VectorSubcoreMesh(
        core_axis_name="c", subcore_axis_name="s",
        num_cores=_NC, num_subcores=_NS)


def _make_gather_combine(d):
    @functools.partial(
        pl.kernel,
        out_type=jax.ShapeDtypeStruct((_E, d), _F32),
        mesh=_sc_mesh(),
        scratch_types=[
            pltpu.VMEM((_C,), jnp.int32),
            pltpu.VMEM((_C,), jnp.int32),
            pltpu.VMEM((_C, d), _F32),
            pltpu.VMEM((_C, d), _F32),
            pltpu.SemaphoreType.DMA,
            pltpu.SemaphoreType.DMA,
        ],
    )
    def k(a_hbm, b_hbm, ia_hbm, ib_hbm, out_hbm, ia_v, ib_v, ra_v, rb_v, sa, sb):
        wid = lax.axis_index("s") * _NC + lax.axis_index("c")
        base = wid * _EPW

        def chunk(j, carry):
            off = base + j * _C
            pltpu.sync_copy(ia_hbm.at[pl.ds(off, _C)], ia_v)
            pltpu.sync_copy(ib_hbm.at[pl.ds(off, _C)], ib_v)
            ca = pltpu.async_copy(a_hbm.at[ia_v], ra_v, sa)
            cb = pltpu.async_copy(b_hbm.at[ib_v], rb_v, sb)
            ca.wait()
            cb.wait()

            def row(r, cc):
                for q in range(d // 16):
                    sl = pl.ds(q * 16, 16)
                    ra_v[r, sl] = ra_v[r, sl] + rb_v[r, sl]
                return cc

            lax.fori_loop(0, _C, row, 0)
            pltpu.sync_copy(ra_v, out_hbm.at[pl.ds(off, _C)])
            return carry

        lax.fori_loop(0, _NCH, chunk, 0)

    return k


def _make_scatter_add():
    @functools.partial(
        pl.kernel,
        out_type=jax.ShapeDtypeStruct((_NC, _NP, _H), _F32),
        mesh=_sc_mesh(),
        scratch_types=[
            pltpu.VMEM((_C,), jnp.int32),
            pltpu.VMEM((_C, _H), _F32),
            pltpu.VMEM_SHARED((_NP, _H), _F32),
        ],
    )
    def k(e_hbm, idx_hbm, out_hbm, ib_v, ev, agg_sh):
        cid = lax.axis_index("c")
        sid = lax.axis_index("s")
        wid = sid * _NC + cid

        def zrow(r, cc):
            for q in range(_H // 16):
                ev[r, pl.ds(q * 16, 16)] = jnp.zeros((16,), _F32)
            return cc

        lax.fori_loop(0, _C, zrow, 0)

        def zcp(q, cc):
            pltpu.sync_copy(ev, agg_sh.at[pl.ds(sid * _RPT + q * _C, _C)])
            return cc

        lax.fori_loop(0, _RPT // _C, zcp, 0)
        plsc.subcore_barrier()

        base = wid * _EPW

        def chunk(j, cc):
            off = base + j * _C
            pltpu.sync_copy(idx_hbm.at[pl.ds(off, _C)], ib_v)
            pltpu.sync_copy(e_hbm.at[pl.ds(off, _C)], ev)
            pltpu.sync_copy(ev, agg_sh.at[ib_v], add=True)
            return cc

        lax.fori_loop(0, _NCH, chunk, 0)
        plsc.subcore_barrier()
        pltpu.sync_copy(agg_sh.at[pl.ds(sid * _RPT, _RPT)],
                        out_hbm.at[cid, pl.ds(sid * _RPT, _RPT)])

    return k


def _ln(o, g, beta):
    mu = jnp.mean(o, axis=-1, keepdims=True)
    var = jnp.mean((o - mu) ** 2, axis=-1, keepdims=True)
    return (o - mu) * lax.rsqrt(var + 1e-5) * g + beta


def _full(shape):
    nd = len(shape)
    return pl.BlockSpec(shape, lambda i: (0,) * nd)


def _node_enc(F, PT3, W1vp, TE, b1n, W2n, b2n, gn, bn, Wxy, b1e, Wd, Ws, b11):
    def body(f_ref, pt_ref, w1_ref, te_ref, b1_ref, w2_ref, b2_ref, g_ref,
             be_ref, wxy_ref, b1e_ref, wd_ref, ws_ref, b11_ref,
             x_ref, a1_ref, b1o_ref, aq_ref, bq_ref):
        f = f_ref[...]
        pt = pt_ref[0, 0, :]
        oh = (pt[:, None] == lax.broadcasted_iota(jnp.int32, (_BLK_N, 8), 1)
              ).astype(_F32)
        x1 = f @ w1_ref[...] + oh @ te_ref[...] + b1_ref[...]
        h = jnp.maximum(x1, 0.0)
        x = _ln(h @ w2_ref[...] + b2_ref[...], g_ref[...], be_ref[...])
        x_ref[...] = x
        q = f[:, 10:12] @ wxy_ref[...]
        aq_ref[...] = q + b1e_ref[...]
        bq_ref[...] = -q
        a1_ref[...] = x @ wd_ref[...] + b11_ref[...]
        b1o_ref[...] = x @ ws_ref[...]

    row = pl.BlockSpec((_BLK_N, _H), lambda i: (i, 0))
    return pl.pallas_call(
        body,
        grid=(_NB_N,),
        in_specs=[
            pl.BlockSpec((_BLK_N, 12), lambda i: (i, 0)),
            pl.BlockSpec((1, 1, _BLK_N), lambda i: (i, 0, 0)),
            _full((12, _H)), _full((8, _H)), _full((1, _H)),
            _full((_H, _H)), _full((1, _H)), _full((1, _H)), _full((1, _H)),
            _full((2, _H)), _full((1, _H)),
            _full((_H, _H)), _full((_H, _H)), _full((1, _H)),
        ],
        out_specs=[row, row, row, row, row],
        out_shape=[jax.ShapeDtypeStruct((_NP, _H), _F32)] * 5,
    )(F, PT3, W1vp, TE, b1n, W2n, b2n, gn, bn, Wxy, b1e, Wd, Ws, b11)


def _edge_enc(G0, Pinv, b1e, w1r2, W2, b2, g, beta):
    def body(g0_ref, pi_ref, b1_ref, w1_ref, w2_ref, b2_ref, g_ref, be_ref,
             out_ref):
        g0 = g0_ref[...]
        delta = (g0 - b1_ref[...]) @ pi_ref[...]
        dist = jnp.sqrt(jnp.sum(delta * delta, axis=-1, keepdims=True))
        h = jnp.maximum(g0 + dist * w1_ref[...], 0.0)
        out_ref[...] = _ln(h @ w2_ref[...] + b2_ref[...], g_ref[...], be_ref[...])

    row = pl.BlockSpec((_BLK_E, _H), lambda i: (i, 0))
    return pl.pallas_call(
        body,
        grid=(_NB_E,),
        in_specs=[
            row, _full((_H, 2)), _full((1, _H)),
            _full((1, _H)), _full((_H, _H)), _full((1, _H)),
            _full((1, _H)), _full((1, _H)),
        ],
        out_specs=row,
        out_shape=jax.ShapeDtypeStruct((_E, _H), _F32),
    )(G0, Pinv, b1e, w1r2, W2, b2, g, beta)


def _edge_block(G, e, W1e, W2, b2, g, beta):
    def body(g_ref, e_ref, w1_ref, w2_ref, b2_ref, g_ln, be_ref, out_ref):
        h = jnp.maximum(g_ref[...] + e_ref[...] @ w1_ref[...], 0.0)
        out_ref[...] = _ln(h @ w2_ref[...] + b2_ref[...], g_ln[...], be_ref[...])

    row = pl.BlockSpec((_BLK_E, _H), lambda i: (i, 0))
    return pl.pallas_call(
        body,
        grid=(_NB_E,),
        in_specs=[row, row, _full((_H, _H)), _full((_H, _H)),
                  _full((1, _H)), _full((1, _H)), _full((1, _H))],
        out_specs=row,
        out_shape=jax.ShapeDtypeStruct((_E, _H), _F32),
    )(G, e, W1e, W2, b2, g, beta)


def _node_update(x, agg2, W1x, W1a, b1, W2, b2, g, beta, Wd, Ws, b1n):
    def body(x_ref, a_ref, w1x_ref, w1a_ref, b1_ref, w2_ref, b2_ref, g_ref,
             be_ref, wd_ref, ws_ref, b1n_ref, xn_ref, an_ref, bn_ref):
        x0 = x_ref[...]
        agg = a_ref[0] + a_ref[1]
        h = jnp.maximum(x0 @ w1x_ref[...] + agg @ w1a_ref[...] + b1_ref[...], 0.0)
        xn = x0 + _ln(h @ w2_ref[...] + b2_ref[...], g_ref[...], be_ref[...])
        xn_ref[...] = xn
        an_ref[...] = xn @ wd_ref[...] + b1n_ref[...]
        bn_ref[...] = xn @ ws_ref[...]

    row = pl.BlockSpec((_BLK_N, _H), lambda i: (i, 0))
    return pl.pallas_call(
        body,
        grid=(_NB_N,),
        in_specs=[
            row, pl.BlockSpec((_NC, _BLK_N, _H), lambda i: (0, i, 0)),
            _full((_H, _H)), _full((_H, _H)), _full((1, _H)),
            _full((_H, _H)), _full((1, _H)), _full((1, _H)), _full((1, _H)),
            _full((_H, _H)), _full((_H, _H)), _full((1, _H)),
        ],
        out_specs=[row, row, row],
        out_shape=[jax.ShapeDtypeStruct((_NP, _H), _F32)] * 3,
    )(x, agg2, W1x, W1a, b1, W2, b2, g, beta, Wd, Ws, b1n)


def _node_final(x, agg2, W1x, W1a, b1, W2, b2, g, beta,
                W1m, b1m, W2m, b2m, W1v, b1v, W2v, b2v):
    def body(x_ref, a_ref, w1x_ref, w1a_ref, b1_ref, w2_ref, b2_ref, g_ref,
             be_ref, w1m_ref, b1m_ref, w2m_ref, b2m_ref,
             w1v_ref, b1v_ref, w2v_ref, b2v_ref, mu_ref, kl_ref):
        i = pl.program_id(0)
        x0 = x_ref[...]
        agg = a_ref[0] + a_ref[1]
        h = jnp.maximum(x0 @ w1x_ref[...] + agg @ w1a_ref[...] + b1_ref[...], 0.0)
        xn = x0 + _ln(h @ w2_ref[...] + b2_ref[...], g_ref[...], be_ref[...])
        hm = jnp.maximum(xn @ w1m_ref[...] + b1m_ref[...], 0.0)
        mu = hm @ w2m_ref[...] + b2m_ref[...]
        mu_ref[...] = mu
        hv = jnp.maximum(xn @ w1v_ref[...] + b1v_ref[...], 0.0)
        lv = jnp.clip(hv @ w2v_ref[...] + b2v_ref[...], -10.0, 4.0)
        sig2 = jnp.exp(lv)
        s = jnp.sum(0.5 * (mu * mu + sig2 - lv - 1.0), axis=-1, keepdims=True)
        gidx = i * _BLK_N + lax.broadcasted_iota(jnp.int32, (_BLK_N, 1), 0)
        mask = (gidx < _N).astype(_F32)
        part = jnp.sum(s * mask) * (1.0 / _N)

        @pl.when(i == 0)
        def _():
            kl_ref[...] = jnp.zeros((1, 1), _F32)

        kl_ref[...] = kl_ref[...] + part

    row = pl.BlockSpec((_BLK_N, _H), lambda i: (i, 0))
    return pl.pallas_call(
        body,
        grid=(_NB_N,),
        in_specs=[
            row, pl.BlockSpec((_NC, _BLK_N, _H), lambda i: (0, i, 0)),
            _full((_H, _H)), _full((_H, _H)), _full((1, _H)),
            _full((_H, _H)), _full((1, _H)), _full((1, _H)), _full((1, _H)),
            _full((_H, _H)), _full((1, _H)), _full((_H, 2)), _full((1, 2)),
            _full((_H, _H)), _full((1, _H)), _full((_H, 2)), _full((1, 2)),
        ],
        out_specs=[pl.BlockSpec((_BLK_N, 2), lambda i: (i, 0)),
                   pl.BlockSpec((1, 1), lambda i: (0, 0))],
        out_shape=[jax.ShapeDtypeStruct((_NP, 2), _F32),
                   jax.ShapeDtypeStruct((1, 1), _F32)],
    )(x, agg2, W1x, W1a, b1, W2, b2, g, beta,
      W1m, b1m, W2m, b2m, W1v, b1v, W2v, b2v)


_gather128 = _make_gather_combine(_H)
_scatter = _make_scatter_add()


def _r(v):
    return v.reshape(1, -1)


def kernel(pos, vel_history, particle_type, edge_index, params):
    p = params
    ne, ee = p["node_enc"], p["edge_enc"]
    blk0, blk1 = p["blocks"][0], p["blocks"][1]

    W1n = ne["W1"]
    W1vp = W1n[:12].at[8:10].add(W1n[12:14])
    TE = p["type_embed"] @ W1n[14:30]
    Wxy, w1r2 = ee["W1"][:2], _r(ee["W1"][2])
    gram = Wxy @ Wxy.T
    a, b, c = gram[0, 0], gram[0, 1], gram[1, 1]
    det = a * c - b * b
    gram_inv = jnp.stack([jnp.stack([c, -b]), jnp.stack([-b, a])]) / det
    Pinv = Wxy.T @ gram_inv

    def esplit(b):
        w = b["edge_mlp"]["W1"]
        return w[:_H], w[_H:2 * _H], w[2 * _H:]

    Wd0, Ws0, We0 = esplit(blk0)
    Wd1, Ws1, We1 = esplit(blk1)

    def nsplit(b):
        w = b["node_mlp"]["W1"]
        return w[:_H], w[_H:]

    Wx0, Wa0 = nsplit(blk0)
    Wx1, Wa1 = nsplit(blk1)

    vel_flat = vel_history.reshape(_N, -1)
    F = jnp.zeros((_NP, 12), _F32).at[:_N].set(
        jnp.concatenate([vel_flat, pos], axis=1))
    PT3 = jnp.zeros((_NP,), jnp.int32).at[:_N].set(
        particle_type.astype(jnp.int32)).reshape(_NB_N, 1, _BLK_N)
    src = edge_index[0].astype(jnp.int32)
    dst = edge_index[1].astype(jnp.int32)

    x, A, B, Aq, Bq = _node_enc(
        F, PT3, W1vp, TE, _r(ne["b1"]), ne["W2"], _r(ne["b2"]),
        _r(ne["g"]), _r(ne["beta"]), Wxy, _r(ee["b1"]), Wd0, Ws0,
        _r(blk0["edge_mlp"]["b1"]))
    G0 = _gather128(Aq, Bq, dst, src)
    e = _edge_enc(G0, Pinv, _r(ee["b1"]), w1r2, ee["W2"], _r(ee["b2"]),
                  _r(ee["g"]), _r(ee["beta"]))

    em0, nm0 = blk0["edge_mlp"], blk0["node_mlp"]
    G = _gather128(A, B, dst, src)
    e = _edge_block(G, e, We0, em0["W2"], _r(em0["b2"]), _r(em0["g"]),
                    _r(em0["beta"]))
    agg2 = _scatter(e, dst)
    x, A, B = _node_update(
        x, agg2, Wx0, Wa0, _r(nm0["b1"]), nm0["W2"], _r(nm0["b2"]),
        _r(nm0["g"]), _r(nm0["beta"]), Wd1, Ws1, _r(blk1["edge_mlp"]["b1"]))

    em1, nm1 = blk1["edge_mlp"], blk1["node_mlp"]
    G = _gather128(A, B, dst, src)
    e = _edge_block(G, e, We1, em1["W2"], _r(em1["b2"]), _r(em1["g"]),
                    _r(em1["beta"]))
    agg2 = _scatter(e, dst)
    mh, vh = p["mu_head"], p["logv_head"]
    mu, kl = _node_final(
        x, agg2, Wx1, Wa1, _r(nm1["b1"]), nm1["W2"], _r(nm1["b2"]),
        _r(nm1["g"]), _r(nm1["beta"]),
        mh["W1"], _r(mh["b1"]), mh["W2"], _r(mh["b2"]),
        vh["W1"], _r(vh["b1"]), vh["W2"], _r(vh["b2"]))

    return mu[:_N], kl.reshape(())


# speedup vs baseline: 2.8395x; 1.2303x over previous
import functools

import jax
import jax.numpy as jnp
from jax import lax
from jax.experimental import pallas as pl
from jax.experimental.pallas import tpu as pltpu
from jax.experimental.pallas import tpu_sc as plsc

_N = 10000
_E = 320000
_H = 128
_NP = 10240
_BLK_N = 512
_NB_N = _NP // _BLK_N
_BLK_E = 512
_NB_E = _E // _BLK_E

_NC = 2
_NS = 16
_NW = _NC * _NS
_EPW = _E // _NW
_C = 80
_NCH = _EPW // _C
_RPT = _NP // _NS

_F32 = jnp.float32


def _sc_mesh():
    return plsc.VectorSubcoreMesh(
        core_axis_name="c", subcore_axis_name="s",
        num_cores=_NC, num_subcores=_NS)


def _make_gather_combine(d):
    @functools.partial(
        pl.kernel,
        out_type=jax.ShapeDtypeStruct((_E, d), _F32),
        mesh=_sc_mesh(),
        scratch_types=[
            pltpu.VMEM((_EPW,), jnp.int32),
            pltpu.VMEM((_EPW,), jnp.int32),
            pltpu.VMEM((2, _C, d), _F32),
            pltpu.VMEM((2, _C, d), _F32),
            pltpu.SemaphoreType.DMA, pltpu.SemaphoreType.DMA,
            pltpu.SemaphoreType.DMA, pltpu.SemaphoreType.DMA,
            pltpu.SemaphoreType.DMA, pltpu.SemaphoreType.DMA,
        ],
    )
    def k(a_hbm, b_hbm, ia_hbm, ib_hbm, out_hbm, ia_v, ib_v, ra_v, rb_v,
          sa0, sa1, sb0, sb1, sw0, sw1):
        sa = (sa0, sa1)
        sb = (sb0, sb1)
        sw = (sw0, sw1)
        wid = lax.axis_index("s") * _NC + lax.axis_index("c")
        base = wid * _EPW
        pltpu.sync_copy(ia_hbm.at[pl.ds(base, _EPW)], ia_v)
        pltpu.sync_copy(ib_hbm.at[pl.ds(base, _EPW)], ib_v)

        def g_args(j, b):
            return ((a_hbm.at[ia_v.at[pl.ds(j * _C, _C)]], ra_v.at[b], sa[b]),
                    (b_hbm.at[ib_v.at[pl.ds(j * _C, _C)]], rb_v.at[b], sb[b]))

        def fire_gather(j, b):
            for args in g_args(j, b):
                pltpu.async_copy(*args)

        def wait_gather(j, b):
            for args in g_args(j, b):
                pltpu.make_async_copy(*args).wait()

        def wb_args(j, b):
            return (ra_v.at[b], out_hbm.at[pl.ds(base + j * _C, _C)], sw[b])

        def step(j, b, bp):
            @pl.when(j < _NCH)
            def _():
                wait_gather(j, b)

                def row(r, cc):
                    for q in range(d // 16):
                        sl = pl.ds(q * 16, 16)
                        ra_v[b, r, sl] = ra_v[b, r, sl] + rb_v[b, r, sl]
                    return cc

                lax.fori_loop(0, _C, row, 0)
                pltpu.async_copy(*wb_args(j, b))

                @pl.when(j + 1 < _NCH)
                def _():
                    @pl.when(j >= 1)
                    def _():
                        pltpu.make_async_copy(*wb_args(j - 1, bp)).wait()

                    fire_gather(j + 1, bp)

        fire_gather(0, 0)

        def body(jj, carry):
            step(2 * jj, 0, 1)
            step(2 * jj + 1, 1, 0)
            return carry

        lax.fori_loop(0, (_NCH + 1) // 2, body, 0)
        pltpu.make_async_copy(*wb_args(_NCH - 2, (_NCH - 2) % 2)).wait()
        pltpu.make_async_copy(*wb_args(_NCH - 1, (_NCH - 1) % 2)).wait()

    return k


def _make_scatter_add():
    @functools.partial(
        pl.kernel,
        out_type=jax.ShapeDtypeStruct((_NC, _NP, _H), _F32),
        mesh=_sc_mesh(),
        scratch_types=[
            pltpu.VMEM((2, _C), jnp.int32),
            pltpu.VMEM((2, _C, _H), _F32),
            pltpu.VMEM_SHARED((_NP, _H), _F32),
            pltpu.SemaphoreType.DMA, pltpu.SemaphoreType.DMA,
            pltpu.SemaphoreType.DMA, pltpu.SemaphoreType.DMA,
        ],
    )
    def k(e_hbm, idx_hbm, out_hbm, ib_v, ev, agg_sh, si0, si1, se0, se1):
        si = (si0, si1)
        se = (se0, se1)
        cid = lax.axis_index("c")
        sid = lax.axis_index("s")
        wid = sid * _NC + cid

        def zrow(r, cc):
            for q in range(_H // 16):
                ev[0, r, pl.ds(q * 16, 16)] = jnp.zeros((16,), _F32)
            return cc

        lax.fori_loop(0, _C, zrow, 0)

        def zcp(q, cc):
            pltpu.sync_copy(ev.at[0], agg_sh.at[pl.ds(sid * _RPT + q * _C, _C)])
            return cc

        lax.fori_loop(0, _RPT // _C, zcp, 0)
        plsc.subcore_barrier()

        base = wid * _EPW

        def ld_args(j, b):
            off = base + j * _C
            return ((idx_hbm.at[pl.ds(off, _C)], ib_v.at[b], si[b]),
                    (e_hbm.at[pl.ds(off, _C)], ev.at[b], se[b]))

        def fire_loads(j, b):
            for args in ld_args(j, b):
                pltpu.async_copy(*args)

        def step(j, b):
            @pl.when(j < _NCH)
            def _():
                for args in ld_args(j, b):
                    pltpu.make_async_copy(*args).wait()
                pltpu.sync_copy(ev.at[b], agg_sh.at[ib_v.at[b]], add=True)

                @pl.when(j + 2 < _NCH)
                def _():
                    fire_loads(j + 2, b)

        fire_loads(0, 0)
        fire_loads(1, 1)

        def body(jj, carry):
            step(2 * jj, 0)
            step(2 * jj + 1, 1)
            return carry

        lax.fori_loop(0, (_NCH + 1) // 2, body, 0)
        plsc.subcore_barrier()
        pltpu.sync_copy(agg_sh.at[pl.ds(sid * _RPT, _RPT)],
                        out_hbm.at[cid, pl.ds(sid * _RPT, _RPT)])

    return k


def _ln(o, g, beta):
    mu = jnp.mean(o, axis=-1, keepdims=True)
    var = jnp.mean((o - mu) ** 2, axis=-1, keepdims=True)
    return (o - mu) * lax.rsqrt(var + 1e-5) * g + beta


def _full(shape):
    nd = len(shape)
    return pl.BlockSpec(shape, lambda i: (0,) * nd)


def _node_enc(F, PT3, W1vp, TE, b1n, W2n, b2n, gn, bn, Wxy, b1e, Wd, Ws, b11):
    def body(f_ref, pt_ref, w1_ref, te_ref, b1_ref, w2_ref, b2_ref, g_ref,
             be_ref, wxy_ref, b1e_ref, wd_ref, ws_ref, b11_ref,
             x_ref, a1_ref, b1o_ref, aq_ref, bq_ref):
        f = f_ref[...]
        pt = pt_ref[0, 0, :]
        oh = (pt[:, None] == lax.broadcasted_iota(jnp.int32, (_BLK_N, 8), 1)
              ).astype(_F32)
        x1 = f @ w1_ref[...] + oh @ te_ref[...] + b1_ref[...]
        h = jnp.maximum(x1, 0.0)
        x = _ln(h @ w2_ref[...] + b2_ref[...], g_ref[...], be_ref[...])
        x_ref[...] = x
        q = f[:, 10:12] @ wxy_ref[...]
        aq_ref[...] = q + b1e_ref[...]
        bq_ref[...] = -q
        a1_ref[...] = x @ wd_ref[...] + b11_ref[...]
        b1o_ref[...] = x @ ws_ref[...]

    row = pl.BlockSpec((_BLK_N, _H), lambda i: (i, 0))
    return pl.pallas_call(
        body,
        grid=(_NB_N,),
        in_specs=[
            pl.BlockSpec((_BLK_N, 12), lambda i: (i, 0)),
            pl.BlockSpec((1, 1, _BLK_N), lambda i: (i, 0, 0)),
            _full((12, _H)), _full((8, _H)), _full((1, _H)),
            _full((_H, _H)), _full((1, _H)), _full((1, _H)), _full((1, _H)),
            _full((2, _H)), _full((1, _H)),
            _full((_H, _H)), _full((_H, _H)), _full((1, _H)),
        ],
        out_specs=[row, row, row, row, row],
        out_shape=[jax.ShapeDtypeStruct((_NP, _H), _F32)] * 5,
    )(F, PT3, W1vp, TE, b1n, W2n, b2n, gn, bn, Wxy, b1e, Wd, Ws, b11)


def _edge_enc(G0, Pinv, b1e, w1r2, W2, b2, g, beta):
    def body(g0_ref, pi_ref, b1_ref, w1_ref, w2_ref, b2_ref, g_ref, be_ref,
             out_ref):
        g0 = g0_ref[...]
        delta = (g0 - b1_ref[...]) @ pi_ref[...]
        dist = jnp.sqrt(jnp.sum(delta * delta, axis=-1, keepdims=True))
        h = jnp.maximum(g0 + dist * w1_ref[...], 0.0)
        out_ref[...] = _ln(h @ w2_ref[...] + b2_ref[...], g_ref[...], be_ref[...])

    row = pl.BlockSpec((_BLK_E, _H), lambda i: (i, 0))
    return pl.pallas_call(
        body,
        grid=(_NB_E,),
        in_specs=[
            row, _full((_H, 2)), _full((1, _H)),
            _full((1, _H)), _full((_H, _H)), _full((1, _H)),
            _full((1, _H)), _full((1, _H)),
        ],
        out_specs=row,
        out_shape=jax.ShapeDtypeStruct((_E, _H), _F32),
    )(G0, Pinv, b1e, w1r2, W2, b2, g, beta)


def _edge_block(G, e, W1e, W2, b2, g, beta):
    def body(g_ref, e_ref, w1_ref, w2_ref, b2_ref, g_ln, be_ref, out_ref):
        h = jnp.maximum(g_ref[...] + e_ref[...] @ w1_ref[...], 0.0)
        out_ref[...] = _ln(h @ w2_ref[...] + b2_ref[...], g_ln[...], be_ref[...])

    row = pl.BlockSpec((_BLK_E, _H), lambda i: (i, 0))
    return pl.pallas_call(
        body,
        grid=(_NB_E,),
        in_specs=[row, row, _full((_H, _H)), _full((_H, _H)),
                  _full((1, _H)), _full((1, _H)), _full((1, _H))],
        out_specs=row,
        out_shape=jax.ShapeDtypeStruct((_E, _H), _F32),
    )(G, e, W1e, W2, b2, g, beta)


def _node_update(x, agg2, W1x, W1a, b1, W2, b2, g, beta, Wd, Ws, b1n):
    def body(x_ref, a_ref, w1x_ref, w1a_ref, b1_ref, w2_ref, b2_ref, g_ref,
             be_ref, wd_ref, ws_ref, b1n_ref, xn_ref, an_ref, bn_ref):
        x0 = x_ref[...]
        agg = a_ref[0] + a_ref[1]
        h = jnp.maximum(x0 @ w1x_ref[...] + agg @ w1a_ref[...] + b1_ref[...], 0.0)
        xn = x0 + _ln(h @ w2_ref[...] + b2_ref[...], g_ref[...], be_ref[...])
        xn_ref[...] = xn
        an_ref[...] = xn @ wd_ref[...] + b1n_ref[...]
        bn_ref[...] = xn @ ws_ref[...]

    row = pl.BlockSpec((_BLK_N, _H), lambda i: (i, 0))
    return pl.pallas_call(
        body,
        grid=(_NB_N,),
        in_specs=[
            row, pl.BlockSpec((_NC, _BLK_N, _H), lambda i: (0, i, 0)),
            _full((_H, _H)), _full((_H, _H)), _full((1, _H)),
            _full((_H, _H)), _full((1, _H)), _full((1, _H)), _full((1, _H)),
            _full((_H, _H)), _full((_H, _H)), _full((1, _H)),
        ],
        out_specs=[row, row, row],
        out_shape=[jax.ShapeDtypeStruct((_NP, _H), _F32)] * 3,
    )(x, agg2, W1x, W1a, b1, W2, b2, g, beta, Wd, Ws, b1n)


def _node_final(x, agg2, W1x, W1a, b1, W2, b2, g, beta,
                W1m, b1m, W2m, b2m, W1v, b1v, W2v, b2v):
    def body(x_ref, a_ref, w1x_ref, w1a_ref, b1_ref, w2_ref, b2_ref, g_ref,
             be_ref, w1m_ref, b1m_ref, w2m_ref, b2m_ref,
             w1v_ref, b1v_ref, w2v_ref, b2v_ref, mu_ref, kl_ref):
        i = pl.program_id(0)
        x0 = x_ref[...]
        agg = a_ref[0] + a_ref[1]
        h = jnp.maximum(x0 @ w1x_ref[...] + agg @ w1a_ref[...] + b1_ref[...], 0.0)
        xn = x0 + _ln(h @ w2_ref[...] + b2_ref[...], g_ref[...], be_ref[...])
        hm = jnp.maximum(xn @ w1m_ref[...] + b1m_ref[...], 0.0)
        mu = hm @ w2m_ref[...] + b2m_ref[...]
        mu_ref[...] = mu
        hv = jnp.maximum(xn @ w1v_ref[...] + b1v_ref[...], 0.0)
        lv = jnp.clip(hv @ w2v_ref[...] + b2v_ref[...], -10.0, 4.0)
        sig2 = jnp.exp(lv)
        s = jnp.sum(0.5 * (mu * mu + sig2 - lv - 1.0), axis=-1, keepdims=True)
        gidx = i * _BLK_N + lax.broadcasted_iota(jnp.int32, (_BLK_N, 1), 0)
        mask = (gidx < _N).astype(_F32)
        part = jnp.sum(s * mask) * (1.0 / _N)

        @pl.when(i == 0)
        def _():
            kl_ref[...] = jnp.zeros((1, 1), _F32)

        kl_ref[...] = kl_ref[...] + part

    row = pl.BlockSpec((_BLK_N, _H), lambda i: (i, 0))
    return pl.pallas_call(
        body,
        grid=(_NB_N,),
        in_specs=[
            row, pl.BlockSpec((_NC, _BLK_N, _H), lambda i: (0, i, 0)),
            _full((_H, _H)), _full((_H, _H)), _full((1, _H)),
            _full((_H, _H)), _full((1, _H)), _full((1, _H)), _full((1, _H)),
            _full((_H, _H)), _full((1, _H)), _full((_H, 2)), _full((1, 2)),
            _full((_H, _H)), _full((1, _H)), _full((_H, 2)), _full((1, 2)),
        ],
        out_specs=[pl.BlockSpec((_BLK_N, 2), lambda i: (i, 0)),
                   pl.BlockSpec((1, 1), lambda i: (0, 0))],
        out_shape=[jax.ShapeDtypeStruct((_NP, 2), _F32),
                   jax.ShapeDtypeStruct((1, 1), _F32)],
    )(x, agg2, W1x, W1a, b1, W2, b2, g, beta,
      W1m, b1m, W2m, b2m, W1v, b1v, W2v, b2v)


_gather128 = _make_gather_combine(_H)
_scatter = _make_scatter_add()


def _r(v):
    return v.reshape(1, -1)


def kernel(pos, vel_history, particle_type, edge_index, params):
    p = params
    ne, ee = p["node_enc"], p["edge_enc"]
    blk0, blk1 = p["blocks"][0], p["blocks"][1]

    W1n = ne["W1"]
    W1vp = W1n[:12].at[8:10].add(W1n[12:14])
    TE = p["type_embed"] @ W1n[14:30]
    Wxy, w1r2 = ee["W1"][:2], _r(ee["W1"][2])
    gram = Wxy @ Wxy.T
    a, b, c = gram[0, 0], gram[0, 1], gram[1, 1]
    det = a * c - b * b
    gram_inv = jnp.stack([jnp.stack([c, -b]), jnp.stack([-b, a])]) / det
    Pinv = Wxy.T @ gram_inv

    def esplit(b):
        w = b["edge_mlp"]["W1"]
        return w[:_H], w[_H:2 * _H], w[2 * _H:]

    Wd0, Ws0, We0 = esplit(blk0)
    Wd1, Ws1, We1 = esplit(blk1)

    def nsplit(b):
        w = b["node_mlp"]["W1"]
        return w[:_H], w[_H:]

    Wx0, Wa0 = nsplit(blk0)
    Wx1, Wa1 = nsplit(blk1)

    vel_flat = vel_history.reshape(_N, -1)
    F = jnp.zeros((_NP, 12), _F32).at[:_N].set(
        jnp.concatenate([vel_flat, pos], axis=1))
    PT3 = jnp.zeros((_NP,), jnp.int32).at[:_N].set(
        particle_type.astype(jnp.int32)).reshape(_NB_N, 1, _BLK_N)
    src = edge_index[0].astype(jnp.int32)
    dst = edge_index[1].astype(jnp.int32)

    x, A, B, Aq, Bq = _node_enc(
        F, PT3, W1vp, TE, _r(ne["b1"]), ne["W2"], _r(ne["b2"]),
        _r(ne["g"]), _r(ne["beta"]), Wxy, _r(ee["b1"]), Wd0, Ws0,
        _r(blk0["edge_mlp"]["b1"]))
    G0 = _gather128(Aq, Bq, dst, src)
    e = _edge_enc(G0, Pinv, _r(ee["b1"]), w1r2, ee["W2"], _r(ee["b2"]),
                  _r(ee["g"]), _r(ee["beta"]))

    em0, nm0 = blk0["edge_mlp"], blk0["node_mlp"]
    G = _gather128(A, B, dst, src)
    e = _edge_block(G, e, We0, em0["W2"], _r(em0["b2"]), _r(em0["g"]),
                    _r(em0["beta"]))
    agg2 = _scatter(e, dst)
    x, A, B = _node_update(
        x, agg2, Wx0, Wa0, _r(nm0["b1"]), nm0["W2"], _r(nm0["b2"]),
        _r(nm0["g"]), _r(nm0["beta"]), Wd1, Ws1, _r(blk1["edge_mlp"]["b1"]))

    em1, nm1 = blk1["edge_mlp"], blk1["node_mlp"]
    G = _gather128(A, B, dst, src)
    e = _edge_block(G, e, We1, em1["W2"], _r(em1["b2"]), _r(em1["g"]),
                    _r(em1["beta"]))
    agg2 = _scatter(e, dst)
    mh, vh = p["mu_head"], p["logv_head"]
    mu, kl = _node_final(
        x, agg2, Wx1, Wa1, _r(nm1["b1"]), nm1["W2"], _r(nm1["b2"]),
        _r(nm1["g"]), _r(nm1["beta"]),
        mh["W1"], _r(mh["b1"]), mh["W2"], _r(mh["b2"]),
        vh["W1"], _r(vh["b1"]), vh["W2"], _r(vh["b2"]))

    return mu[:_N], kl.reshape(())


# speedup vs baseline: 3.7306x; 1.3138x over previous
import functools

import jax
import jax.numpy as jnp
from jax import lax
from jax.experimental import pallas as pl
from jax.experimental.pallas import tpu as pltpu
from jax.experimental.pallas import tpu_sc as plsc

_N = 10000
_E = 320000
_H = 128
_NP = 10240
_BLK_N = 512
_NB_N = _NP // _BLK_N
_BLK_E = 2000
_NB_E = _E // _BLK_E

_NC = 2
_NS = 16
_NW = _NC * _NS
_EPW = _E // _NW
_C = 80
_NCH = _EPW // _C
_RPT = _NP // _NS

_F32 = jnp.float32


def _sc_mesh():
    return plsc.VectorSubcoreMesh(
        core_axis_name="c", subcore_axis_name="s",
        num_cores=_NC, num_subcores=_NS)


def _make_gather_combine(d):
    @functools.partial(
        pl.kernel,
        out_type=jax.ShapeDtypeStruct((_E, d), _F32),
        mesh=_sc_mesh(),
        scratch_types=[
            pltpu.VMEM((_EPW,), jnp.int32),
            pltpu.VMEM((_EPW,), jnp.int32),
            pltpu.VMEM((2, _C, d), _F32),
            pltpu.VMEM((2, _C, d), _F32),
            pltpu.SemaphoreType.DMA, pltpu.SemaphoreType.DMA,
            pltpu.SemaphoreType.DMA, pltpu.SemaphoreType.DMA,
            pltpu.SemaphoreType.DMA, pltpu.SemaphoreType.DMA,
        ],
    )
    def k(a_hbm, b_hbm, ia_hbm, ib_hbm, out_hbm, ia_v, ib_v, ra_v, rb_v,
          sa0, sa1, sb0, sb1, sw0, sw1):
        sa = (sa0, sa1)
        sb = (sb0, sb1)
        sw = (sw0, sw1)
        wid = lax.axis_index("s") * _NC + lax.axis_index("c")
        base = wid * _EPW
        pltpu.sync_copy(ia_hbm.at[pl.ds(base, _EPW)], ia_v)
        pltpu.sync_copy(ib_hbm.at[pl.ds(base, _EPW)], ib_v)

        def g_args(j, b):
            return ((a_hbm.at[ia_v.at[pl.ds(j * _C, _C)]], ra_v.at[b], sa[b]),
                    (b_hbm.at[ib_v.at[pl.ds(j * _C, _C)]], rb_v.at[b], sb[b]))

        def fire_gather(j, b):
            for args in g_args(j, b):
                pltpu.async_copy(*args)

        def wait_gather(j, b):
            for args in g_args(j, b):
                pltpu.make_async_copy(*args).wait()

        def wb_args(j, b):
            return (ra_v.at[b], out_hbm.at[pl.ds(base + j * _C, _C)], sw[b])

        def step(j, b, bp):
            @pl.when(j < _NCH)
            def _():
                wait_gather(j, b)

                def row(r, cc):
                    for q in range(d // 16):
                        sl = pl.ds(q * 16, 16)
                        ra_v[b, r, sl] = ra_v[b, r, sl] + rb_v[b, r, sl]
                    return cc

                lax.fori_loop(0, _C, row, 0)
                pltpu.async_copy(*wb_args(j, b))

                @pl.when(j + 1 < _NCH)
                def _():
                    @pl.when(j >= 1)
                    def _():
                        pltpu.make_async_copy(*wb_args(j - 1, bp)).wait()

                    fire_gather(j + 1, bp)

        fire_gather(0, 0)

        def body(jj, carry):
            step(2 * jj, 0, 1)
            step(2 * jj + 1, 1, 0)
            return carry

        lax.fori_loop(0, (_NCH + 1) // 2, body, 0)
        pltpu.make_async_copy(*wb_args(_NCH - 2, (_NCH - 2) % 2)).wait()
        pltpu.make_async_copy(*wb_args(_NCH - 1, (_NCH - 1) % 2)).wait()

    return k


def _make_scatter_add():
    @functools.partial(
        pl.kernel,
        out_type=jax.ShapeDtypeStruct((_NC, _NP, _H), _F32),
        mesh=_sc_mesh(),
        scratch_types=[
            pltpu.VMEM((2, _C), jnp.int32),
            pltpu.VMEM((2, _C, _H), _F32),
            pltpu.VMEM_SHARED((_NP, _H), _F32),
            pltpu.SemaphoreType.DMA, pltpu.SemaphoreType.DMA,
            pltpu.SemaphoreType.DMA, pltpu.SemaphoreType.DMA,
        ],
    )
    def k(e_hbm, idx_hbm, out_hbm, ib_v, ev, agg_sh, si0, si1, se0, se1):
        si = (si0, si1)
        se = (se0, se1)
        cid = lax.axis_index("c")
        sid = lax.axis_index("s")
        wid = sid * _NC + cid

        def zrow(r, cc):
            for q in range(_H // 16):
                ev[0, r, pl.ds(q * 16, 16)] = jnp.zeros((16,), _F32)
            return cc

        lax.fori_loop(0, _C, zrow, 0)

        def zcp(q, cc):
            pltpu.sync_copy(ev.at[0], agg_sh.at[pl.ds(sid * _RPT + q * _C, _C)])
            return cc

        lax.fori_loop(0, _RPT // _C, zcp, 0)
        plsc.subcore_barrier()

        base = wid * _EPW

        def ld_args(j, b):
            off = base + j * _C
            return ((idx_hbm.at[pl.ds(off, _C)], ib_v.at[b], si[b]),
                    (e_hbm.at[pl.ds(off, _C)], ev.at[b], se[b]))

        def fire_loads(j, b):
            for args in ld_args(j, b):
                pltpu.async_copy(*args)

        def step(j, b):
            @pl.when(j < _NCH)
            def _():
                for args in ld_args(j, b):
                    pltpu.make_async_copy(*args).wait()
                pltpu.sync_copy(ev.at[b], agg_sh.at[ib_v.at[b]], add=True)

                @pl.when(j + 2 < _NCH)
                def _():
                    fire_loads(j + 2, b)

        fire_loads(0, 0)
        fire_loads(1, 1)

        def body(jj, carry):
            step(2 * jj, 0)
            step(2 * jj + 1, 1)
            return carry

        lax.fori_loop(0, (_NCH + 1) // 2, body, 0)
        plsc.subcore_barrier()
        pltpu.sync_copy(agg_sh.at[pl.ds(sid * _RPT, _RPT)],
                        out_hbm.at[cid, pl.ds(sid * _RPT, _RPT)])

    return k


def _ln(o, g, beta):
    mu = jnp.mean(o, axis=-1, keepdims=True)
    var = jnp.mean((o - mu) ** 2, axis=-1, keepdims=True)
    return (o - mu) * lax.rsqrt(var + 1e-5) * g + beta


def _full(shape):
    nd = len(shape)
    return pl.BlockSpec(shape, lambda i: (0,) * nd)


def _node_enc(F, PT3, W1vp, TE, b1n, W2n, b2n, gn, bn, Wxy, b1e, Wd, Ws, b11):
    def body(f_ref, pt_ref, w1_ref, te_ref, b1_ref, w2_ref, b2_ref, g_ref,
             be_ref, wxy_ref, b1e_ref, wd_ref, ws_ref, b11_ref,
             x_ref, aw_ref, bw_ref):
        f = f_ref[...]
        pt = pt_ref[0, 0, :]
        oh = (pt[:, None] == lax.broadcasted_iota(jnp.int32, (_BLK_N, 8), 1)
              ).astype(_F32)
        x1 = f @ w1_ref[...] + oh @ te_ref[...] + b1_ref[...]
        h = jnp.maximum(x1, 0.0)
        x = _ln(h @ w2_ref[...] + b2_ref[...], g_ref[...], be_ref[...])
        x_ref[...] = x
        q = f[:, 10:12] @ wxy_ref[...]
        aw_ref[:, :_H] = q + b1e_ref[...]
        bw_ref[:, :_H] = -q
        aw_ref[:, _H:] = x @ wd_ref[...] + b11_ref[...]
        bw_ref[:, _H:] = x @ ws_ref[...]

    row = pl.BlockSpec((_BLK_N, _H), lambda i: (i, 0))
    row2 = pl.BlockSpec((_BLK_N, 2 * _H), lambda i: (i, 0))
    return pl.pallas_call(
        body,
        grid=(_NB_N,),
        in_specs=[
            pl.BlockSpec((_BLK_N, 12), lambda i: (i, 0)),
            pl.BlockSpec((1, 1, _BLK_N), lambda i: (i, 0, 0)),
            _full((12, _H)), _full((8, _H)), _full((1, _H)),
            _full((_H, _H)), _full((1, _H)), _full((1, _H)), _full((1, _H)),
            _full((2, _H)), _full((1, _H)),
            _full((_H, _H)), _full((_H, _H)), _full((1, _H)),
        ],
        out_specs=[row, row2, row2],
        out_shape=[jax.ShapeDtypeStruct((_NP, _H), _F32),
                   jax.ShapeDtypeStruct((_NP, 2 * _H), _F32),
                   jax.ShapeDtypeStruct((_NP, 2 * _H), _F32)],
    )(F, PT3, W1vp, TE, b1n, W2n, b2n, gn, bn, Wxy, b1e, Wd, Ws, b11)


def _edge_block0(GW, Pinv, b1e, w1r2, W2e, b2e, ge, bee, We, W2, b2, g, beta):
    def body(gw_ref, pi_ref, b1_ref, w1_ref, w2e_ref, b2e_ref, ge_ref,
             bee_ref, we_ref, w2_ref, b2_ref, g_ref, be_ref, out_ref):
        g0 = gw_ref[:, :_H]
        delta = (g0 - b1_ref[...]) @ pi_ref[...]
        dist = jnp.sqrt(jnp.sum(delta * delta, axis=-1, keepdims=True))
        h0 = jnp.maximum(g0 + dist * w1_ref[...], 0.0)
        e0 = _ln(h0 @ w2e_ref[...] + b2e_ref[...], ge_ref[...], bee_ref[...])
        h = jnp.maximum(gw_ref[:, _H:] + e0 @ we_ref[...], 0.0)
        out_ref[...] = _ln(h @ w2_ref[...] + b2_ref[...], g_ref[...], be_ref[...])

    row = pl.BlockSpec((_BLK_E, _H), lambda i: (i, 0))
    return pl.pallas_call(
        body,
        grid=(_NB_E,),
        in_specs=[
            pl.BlockSpec((_BLK_E, 2 * _H), lambda i: (i, 0)),
            _full((_H, 2)), _full((1, _H)), _full((1, _H)),
            _full((_H, _H)), _full((1, _H)), _full((1, _H)), _full((1, _H)),
            _full((_H, _H)), _full((_H, _H)),
            _full((1, _H)), _full((1, _H)), _full((1, _H)),
        ],
        out_specs=row,
        out_shape=jax.ShapeDtypeStruct((_E, _H), _F32),
    )(GW, Pinv, b1e, w1r2, W2e, b2e, ge, bee, We, W2, b2, g, beta)


def _edge_block(G, e, W1e, W2, b2, g, beta):
    def body(g_ref, e_ref, w1_ref, w2_ref, b2_ref, g_ln, be_ref, out_ref):
        h = jnp.maximum(g_ref[...] + e_ref[...] @ w1_ref[...], 0.0)
        out_ref[...] = _ln(h @ w2_ref[...] + b2_ref[...], g_ln[...], be_ref[...])

    row = pl.BlockSpec((_BLK_E, _H), lambda i: (i, 0))
    return pl.pallas_call(
        body,
        grid=(_NB_E,),
        in_specs=[row, row, _full((_H, _H)), _full((_H, _H)),
                  _full((1, _H)), _full((1, _H)), _full((1, _H))],
        out_specs=row,
        out_shape=jax.ShapeDtypeStruct((_E, _H), _F32),
    )(G, e, W1e, W2, b2, g, beta)


def _node_update(x, agg2, W1x, W1a, b1, W2, b2, g, beta, Wd, Ws, b1n):
    def body(x_ref, a_ref, w1x_ref, w1a_ref, b1_ref, w2_ref, b2_ref, g_ref,
             be_ref, wd_ref, ws_ref, b1n_ref, xn_ref, an_ref, bn_ref):
        x0 = x_ref[...]
        agg = a_ref[0] + a_ref[1]
        h = jnp.maximum(x0 @ w1x_ref[...] + agg @ w1a_ref[...] + b1_ref[...], 0.0)
        xn = x0 + _ln(h @ w2_ref[...] + b2_ref[...], g_ref[...], be_ref[...])
        xn_ref[...] = xn
        an_ref[...] = xn @ wd_ref[...] + b1n_ref[...]
        bn_ref[...] = xn @ ws_ref[...]

    row = pl.BlockSpec((_BLK_N, _H), lambda i: (i, 0))
    return pl.pallas_call(
        body,
        grid=(_NB_N,),
        in_specs=[
            row, pl.BlockSpec((_NC, _BLK_N, _H), lambda i: (0, i, 0)),
            _full((_H, _H)), _full((_H, _H)), _full((1, _H)),
            _full((_H, _H)), _full((1, _H)), _full((1, _H)), _full((1, _H)),
            _full((_H, _H)), _full((_H, _H)), _full((1, _H)),
        ],
        out_specs=[row, row, row],
        out_shape=[jax.ShapeDtypeStruct((_NP, _H), _F32)] * 3,
    )(x, agg2, W1x, W1a, b1, W2, b2, g, beta, Wd, Ws, b1n)


def _node_final(x, agg2, W1x, W1a, b1, W2, b2, g, beta,
                W1m, b1m, W2m, b2m, W1v, b1v, W2v, b2v):
    def body(x_ref, a_ref, w1x_ref, w1a_ref, b1_ref, w2_ref, b2_ref, g_ref,
             be_ref, w1m_ref, b1m_ref, w2m_ref, b2m_ref,
             w1v_ref, b1v_ref, w2v_ref, b2v_ref, mu_ref, kl_ref):
        i = pl.program_id(0)
        x0 = x_ref[...]
        agg = a_ref[0] + a_ref[1]
        h = jnp.maximum(x0 @ w1x_ref[...] + agg @ w1a_ref[...] + b1_ref[...], 0.0)
        xn = x0 + _ln(h @ w2_ref[...] + b2_ref[...], g_ref[...], be_ref[...])
        hm = jnp.maximum(xn @ w1m_ref[...] + b1m_ref[...], 0.0)
        mu = hm @ w2m_ref[...] + b2m_ref[...]
        mu_ref[...] = mu
        hv = jnp.maximum(xn @ w1v_ref[...] + b1v_ref[...], 0.0)
        lv = jnp.clip(hv @ w2v_ref[...] + b2v_ref[...], -10.0, 4.0)
        sig2 = jnp.exp(lv)
        s = jnp.sum(0.5 * (mu * mu + sig2 - lv - 1.0), axis=-1, keepdims=True)
        gidx = i * _BLK_N + lax.broadcasted_iota(jnp.int32, (_BLK_N, 1), 0)
        mask = (gidx < _N).astype(_F32)
        part = jnp.sum(s * mask) * (1.0 / _N)

        @pl.when(i == 0)
        def _():
            kl_ref[...] = jnp.zeros((1, 1), _F32)

        kl_ref[...] = kl_ref[...] + part

    row = pl.BlockSpec((_BLK_N, _H), lambda i: (i, 0))
    return pl.pallas_call(
        body,
        grid=(_NB_N,),
        in_specs=[
            row, pl.BlockSpec((_NC, _BLK_N, _H), lambda i: (0, i, 0)),
            _full((_H, _H)), _full((_H, _H)), _full((1, _H)),
            _full((_H, _H)), _full((1, _H)), _full((1, _H)), _full((1, _H)),
            _full((_H, _H)), _full((1, _H)), _full((_H, 2)), _full((1, 2)),
            _full((_H, _H)), _full((1, _H)), _full((_H, 2)), _full((1, 2)),
        ],
        out_specs=[pl.BlockSpec((_BLK_N, 2), lambda i: (i, 0)),
                   pl.BlockSpec((1, 1), lambda i: (0, 0))],
        out_shape=[jax.ShapeDtypeStruct((_NP, 2), _F32),
                   jax.ShapeDtypeStruct((1, 1), _F32)],
    )(x, agg2, W1x, W1a, b1, W2, b2, g, beta,
      W1m, b1m, W2m, b2m, W1v, b1v, W2v, b2v)


_gather128 = _make_gather_combine(_H)
_gather256 = _make_gather_combine(2 * _H)
_scatter = _make_scatter_add()


def _r(v):
    return v.reshape(1, -1)


def kernel(pos, vel_history, particle_type, edge_index, params):
    p = params
    ne, ee = p["node_enc"], p["edge_enc"]
    blk0, blk1 = p["blocks"][0], p["blocks"][1]

    W1n = ne["W1"]
    W1vp = W1n[:12].at[8:10].add(W1n[12:14])
    TE = p["type_embed"] @ W1n[14:30]
    Wxy, w1r2 = ee["W1"][:2], _r(ee["W1"][2])
    gram = Wxy @ Wxy.T
    a, b, c = gram[0, 0], gram[0, 1], gram[1, 1]
    det = a * c - b * b
    gram_inv = jnp.stack([jnp.stack([c, -b]), jnp.stack([-b, a])]) / det
    Pinv = Wxy.T @ gram_inv

    def esplit(b):
        w = b["edge_mlp"]["W1"]
        return w[:_H], w[_H:2 * _H], w[2 * _H:]

    Wd0, Ws0, We0 = esplit(blk0)
    Wd1, Ws1, We1 = esplit(blk1)

    def nsplit(b):
        w = b["node_mlp"]["W1"]
        return w[:_H], w[_H:]

    Wx0, Wa0 = nsplit(blk0)
    Wx1, Wa1 = nsplit(blk1)

    vel_flat = vel_history.reshape(_N, -1)
    F = jnp.zeros((_NP, 12), _F32).at[:_N].set(
        jnp.concatenate([vel_flat, pos], axis=1))
    PT3 = jnp.zeros((_NP,), jnp.int32).at[:_N].set(
        particle_type.astype(jnp.int32)).reshape(_NB_N, 1, _BLK_N)
    src = edge_index[0].astype(jnp.int32)
    dst = edge_index[1].astype(jnp.int32)

    em0, nm0 = blk0["edge_mlp"], blk0["node_mlp"]
    x, AW, BW = _node_enc(
        F, PT3, W1vp, TE, _r(ne["b1"]), ne["W2"], _r(ne["b2"]),
        _r(ne["g"]), _r(ne["beta"]), Wxy, _r(ee["b1"]), Wd0, Ws0,
        _r(blk0["edge_mlp"]["b1"]))
    GW = _gather256(AW, BW, dst, src)
    e = _edge_block0(GW, Pinv, _r(ee["b1"]), w1r2, ee["W2"], _r(ee["b2"]),
                     _r(ee["g"]), _r(ee["beta"]), We0, em0["W2"],
                     _r(em0["b2"]), _r(em0["g"]), _r(em0["beta"]))
    agg2 = _scatter(e, dst)
    x, A, B = _node_update(
        x, agg2, Wx0, Wa0, _r(nm0["b1"]), nm0["W2"], _r(nm0["b2"]),
        _r(nm0["g"]), _r(nm0["beta"]), Wd1, Ws1, _r(blk1["edge_mlp"]["b1"]))

    em1, nm1 = blk1["edge_mlp"], blk1["node_mlp"]
    G = _gather128(A, B, dst, src)
    e = _edge_block(G, e, We1, em1["W2"], _r(em1["b2"]), _r(em1["g"]),
                    _r(em1["beta"]))
    agg2 = _scatter(e, dst)
    mh, vh = p["mu_head"], p["logv_head"]
    mu, kl = _node_final(
        x, agg2, Wx1, Wa1, _r(nm1["b1"]), nm1["W2"], _r(nm1["b2"]),
        _r(nm1["g"]), _r(nm1["beta"]),
        mh["W1"], _r(mh["b1"]), mh["W2"], _r(mh["b2"]),
        vh["W1"], _r(vh["b1"]), vh["W2"], _r(vh["b2"]))

    return mu[:_N], kl.reshape(())


# speedup vs baseline: 4.1461x; 1.1114x over previous
import functools

import jax
import jax.numpy as jnp
from jax import lax
from jax.experimental import pallas as pl
from jax.experimental.pallas import tpu as pltpu
from jax.experimental.pallas import tpu_sc as plsc

_N = 10000
_E = 320000
_H = 128
_NP = 10240
_BLK_N = 512
_NB_N = _NP // _BLK_N
_BLK_E = 2000
_NB_E = _E // _BLK_E

_NC = 2
_NS = 16
_NW = _NC * _NS
_EPW = _E // _NW
_C = 80
_NCH = _EPW // _C
_RPT = _NP // _NS

_F32 = jnp.float32


def _sc_mesh():
    return plsc.VectorSubcoreMesh(
        core_axis_name="c", subcore_axis_name="s",
        num_cores=_NC, num_subcores=_NS)


def _make_gather_combine(d):
    @functools.partial(
        pl.kernel,
        out_type=jax.ShapeDtypeStruct((_E, d), _F32),
        mesh=_sc_mesh(),
        scratch_types=[
            pltpu.VMEM((_EPW,), jnp.int32),
            pltpu.VMEM((_EPW,), jnp.int32),
            pltpu.VMEM((2, _C, d), _F32),
            pltpu.VMEM((2, _C, d), _F32),
            pltpu.SemaphoreType.DMA, pltpu.SemaphoreType.DMA,
            pltpu.SemaphoreType.DMA, pltpu.SemaphoreType.DMA,
            pltpu.SemaphoreType.DMA, pltpu.SemaphoreType.DMA,
        ],
    )
    def k(a_hbm, b_hbm, ia_hbm, ib_hbm, out_hbm, ia_v, ib_v, ra_v, rb_v,
          sa0, sa1, sb0, sb1, sw0, sw1):
        sa = (sa0, sa1)
        sb = (sb0, sb1)
        sw = (sw0, sw1)
        wid = lax.axis_index("s") * _NC + lax.axis_index("c")
        base = wid * _EPW
        pltpu.sync_copy(ia_hbm.at[pl.ds(base, _EPW)], ia_v)
        pltpu.sync_copy(ib_hbm.at[pl.ds(base, _EPW)], ib_v)

        def g_args(j, b):
            return ((a_hbm.at[ia_v.at[pl.ds(j * _C, _C)]], ra_v.at[b], sa[b]),
                    (b_hbm.at[ib_v.at[pl.ds(j * _C, _C)]], rb_v.at[b], sb[b]))

        def fire_gather(j, b):
            for args in g_args(j, b):
                pltpu.async_copy(*args)

        def wait_gather(j, b):
            for args in g_args(j, b):
                pltpu.make_async_copy(*args).wait()

        def wb_args(j, b):
            return (ra_v.at[b], out_hbm.at[pl.ds(base + j * _C, _C)], sw[b])

        def step(j, b, bp):
            @pl.when(j < _NCH)
            def _():
                wait_gather(j, b)

                def row(r, cc):
                    for q in range(d // 16):
                        sl = pl.ds(q * 16, 16)
                        ra_v[b, r, sl] = ra_v[b, r, sl] + rb_v[b, r, sl]
                    return cc

                lax.fori_loop(0, _C, row, 0)
                pltpu.async_copy(*wb_args(j, b))

                @pl.when(j + 1 < _NCH)
                def _():
                    @pl.when(j >= 1)
                    def _():
                        pltpu.make_async_copy(*wb_args(j - 1, bp)).wait()

                    fire_gather(j + 1, bp)

        fire_gather(0, 0)

        def body(jj, carry):
            step(2 * jj, 0, 1)
            step(2 * jj + 1, 1, 0)
            return carry

        lax.fori_loop(0, (_NCH + 1) // 2, body, 0)
        pltpu.make_async_copy(*wb_args(_NCH - 2, (_NCH - 2) % 2)).wait()
        pltpu.make_async_copy(*wb_args(_NCH - 1, (_NCH - 1) % 2)).wait()

    return k


def _make_scatter_add():
    @functools.partial(
        pl.kernel,
        out_type=jax.ShapeDtypeStruct((_NC, _NP, _H), _F32),
        mesh=_sc_mesh(),
        scratch_types=[
            pltpu.VMEM((2, _C), jnp.int32),
            pltpu.VMEM((2, _C, _H), _F32),
            pltpu.VMEM_SHARED((_NP, _H), _F32),
            pltpu.SemaphoreType.DMA, pltpu.SemaphoreType.DMA,
            pltpu.SemaphoreType.DMA, pltpu.SemaphoreType.DMA,
        ],
    )
    def k(e_hbm, idx_hbm, out_hbm, ib_v, ev, agg_sh, si0, si1, se0, se1):
        si = (si0, si1)
        se = (se0, se1)
        cid = lax.axis_index("c")
        sid = lax.axis_index("s")
        wid = sid * _NC + cid

        def zrow(r, cc):
            for q in range(_H // 16):
                ev[0, r, pl.ds(q * 16, 16)] = jnp.zeros((16,), _F32)
            return cc

        lax.fori_loop(0, _C, zrow, 0)

        def zcp(q, cc):
            pltpu.sync_copy(ev.at[0], agg_sh.at[pl.ds(sid * _RPT + q * _C, _C)])
            return cc

        lax.fori_loop(0, _RPT // _C, zcp, 0)
        plsc.subcore_barrier()

        base = wid * _EPW

        def ld_args(j, b):
            off = base + j * _C
            return ((idx_hbm.at[pl.ds(off, _C)], ib_v.at[b], si[b]),
                    (e_hbm.at[pl.ds(off, _C)], ev.at[b], se[b]))

        def fire_loads(j, b):
            for args in ld_args(j, b):
                pltpu.async_copy(*args)

        def step(j, b):
            @pl.when(j < _NCH)
            def _():
                for args in ld_args(j, b):
                    pltpu.make_async_copy(*args).wait()
                pltpu.sync_copy(ev.at[b], agg_sh.at[ib_v.at[b]], add=True)

                @pl.when(j + 2 < _NCH)
                def _():
                    fire_loads(j + 2, b)

        fire_loads(0, 0)
        fire_loads(1, 1)

        def body(jj, carry):
            step(2 * jj, 0)
            step(2 * jj + 1, 1)
            return carry

        lax.fori_loop(0, (_NCH + 1) // 2, body, 0)
        plsc.subcore_barrier()
        pltpu.sync_copy(agg_sh.at[pl.ds(sid * _RPT, _RPT)],
                        out_hbm.at[cid, pl.ds(sid * _RPT, _RPT)])

    return k


def _ln(o, g, beta):
    mu = jnp.mean(o, axis=-1, keepdims=True)
    var = jnp.mean((o - mu) ** 2, axis=-1, keepdims=True)
    return (o - mu) * lax.rsqrt(var + 1e-5) * g + beta


def _full(shape):
    nd = len(shape)
    return pl.BlockSpec(shape, lambda i: (0,) * nd)


def _node_enc(F, PT3, W1vp, TE, b1n, W2n, b2n, gn, bn, Wxy, b1e, Wd, Ws, b11):
    def body(f_ref, pt_ref, w1_ref, te_ref, b1_ref, w2_ref, b2_ref, g_ref,
             be_ref, wxy_ref, b1e_ref, wd_ref, ws_ref, b11_ref,
             x_ref, aq_ref, bq_ref, a1_ref, b1o_ref):
        f = f_ref[...]
        pt = pt_ref[0, 0, :]
        oh = (pt[:, None] == lax.broadcasted_iota(jnp.int32, (_BLK_N, 8), 1)
              ).astype(_F32)
        x1 = f @ w1_ref[...] + oh @ te_ref[...] + b1_ref[...]
        h = jnp.maximum(x1, 0.0)
        x = _ln(h @ w2_ref[...] + b2_ref[...], g_ref[...], be_ref[...])
        x_ref[...] = x
        q = f[:, 10:12] @ wxy_ref[...]
        aq_ref[...] = q + b1e_ref[...]
        bq_ref[...] = -q
        a1_ref[...] = x @ wd_ref[...] + b11_ref[...]
        b1o_ref[...] = x @ ws_ref[...]

    row = pl.BlockSpec((_BLK_N, _H), lambda i: (i, 0))
    return pl.pallas_call(
        body,
        grid=(_NB_N,),
        in_specs=[
            pl.BlockSpec((_BLK_N, 12), lambda i: (i, 0)),
            pl.BlockSpec((1, 1, _BLK_N), lambda i: (i, 0, 0)),
            _full((12, _H)), _full((8, _H)), _full((1, _H)),
            _full((_H, _H)), _full((1, _H)), _full((1, _H)), _full((1, _H)),
            _full((2, _H)), _full((1, _H)),
            _full((_H, _H)), _full((_H, _H)), _full((1, _H)),
        ],
        out_specs=[row, row, row, row, row],
        out_shape=[jax.ShapeDtypeStruct((_NP, _H), _F32)] * 5,
    )(F, PT3, W1vp, TE, b1n, W2n, b2n, gn, bn, Wxy, b1e, Wd, Ws, b11)


def _edge_block0(G0, G1, Pinv, b1e, w1r2, W2e, b2e, ge, bee, We, W2, b2, g,
                 beta):
    def body(g0_ref, g1_ref, pi_ref, b1_ref, w1_ref, w2e_ref, b2e_ref, ge_ref,
             bee_ref, we_ref, w2_ref, b2_ref, g_ref, be_ref, out_ref):
        g0 = g0_ref[...]
        delta = (g0 - b1_ref[...]) @ pi_ref[...]
        dist = jnp.sqrt(jnp.sum(delta * delta, axis=-1, keepdims=True))
        h0 = jnp.maximum(g0 + dist * w1_ref[...], 0.0)
        e0 = _ln(h0 @ w2e_ref[...] + b2e_ref[...], ge_ref[...], bee_ref[...])
        h = jnp.maximum(g1_ref[...] + e0 @ we_ref[...], 0.0)
        out_ref[...] = _ln(h @ w2_ref[...] + b2_ref[...], g_ref[...], be_ref[...])

    row = pl.BlockSpec((_BLK_E, _H), lambda i: (i, 0))
    return pl.pallas_call(
        body,
        grid=(_NB_E,),
        in_specs=[
            row, row,
            _full((_H, 2)), _full((1, _H)), _full((1, _H)),
            _full((_H, _H)), _full((1, _H)), _full((1, _H)), _full((1, _H)),
            _full((_H, _H)), _full((_H, _H)),
            _full((1, _H)), _full((1, _H)), _full((1, _H)),
        ],
        out_specs=row,
        out_shape=jax.ShapeDtypeStruct((_E, _H), _F32),
    )(G0, G1, Pinv, b1e, w1r2, W2e, b2e, ge, bee, We, W2, b2, g, beta)


def _edge_block(G, e, W1e, W2, b2, g, beta):
    def body(g_ref, e_ref, w1_ref, w2_ref, b2_ref, g_ln, be_ref, out_ref):
        h = jnp.maximum(g_ref[...] + e_ref[...] @ w1_ref[...], 0.0)
        out_ref[...] = _ln(h @ w2_ref[...] + b2_ref[...], g_ln[...], be_ref[...])

    row = pl.BlockSpec((_BLK_E, _H), lambda i: (i, 0))
    return pl.pallas_call(
        body,
        grid=(_NB_E,),
        in_specs=[row, row, _full((_H, _H)), _full((_H, _H)),
                  _full((1, _H)), _full((1, _H)), _full((1, _H))],
        out_specs=row,
        out_shape=jax.ShapeDtypeStruct((_E, _H), _F32),
    )(G, e, W1e, W2, b2, g, beta)


def _node_update(x, agg2, W1x, W1a, b1, W2, b2, g, beta, Wd, Ws, b1n):
    def body(x_ref, a_ref, w1x_ref, w1a_ref, b1_ref, w2_ref, b2_ref, g_ref,
             be_ref, wd_ref, ws_ref, b1n_ref, xn_ref, an_ref, bn_ref):
        x0 = x_ref[...]
        agg = a_ref[0] + a_ref[1]
        h = jnp.maximum(x0 @ w1x_ref[...] + agg @ w1a_ref[...] + b1_ref[...], 0.0)
        xn = x0 + _ln(h @ w2_ref[...] + b2_ref[...], g_ref[...], be_ref[...])
        xn_ref[...] = xn
        an_ref[...] = xn @ wd_ref[...] + b1n_ref[...]
        bn_ref[...] = xn @ ws_ref[...]

    row = pl.BlockSpec((_BLK_N, _H), lambda i: (i, 0))
    return pl.pallas_call(
        body,
        grid=(_NB_N,),
        in_specs=[
            row, pl.BlockSpec((_NC, _BLK_N, _H), lambda i: (0, i, 0)),
            _full((_H, _H)), _full((_H, _H)), _full((1, _H)),
            _full((_H, _H)), _full((1, _H)), _full((1, _H)), _full((1, _H)),
            _full((_H, _H)), _full((_H, _H)), _full((1, _H)),
        ],
        out_specs=[row, row, row],
        out_shape=[jax.ShapeDtypeStruct((_NP, _H), _F32)] * 3,
    )(x, agg2, W1x, W1a, b1, W2, b2, g, beta, Wd, Ws, b1n)


def _node_final(x, agg2, W1x, W1a, b1, W2, b2, g, beta,
                W1m, b1m, W2m, b2m, W1v, b1v, W2v, b2v):
    def body(x_ref, a_ref, w1x_ref, w1a_ref, b1_ref, w2_ref, b2_ref, g_ref,
             be_ref, w1m_ref, b1m_ref, w2m_ref, b2m_ref,
             w1v_ref, b1v_ref, w2v_ref, b2v_ref, mu_ref, kl_ref):
        i = pl.program_id(0)
        x0 = x_ref[...]
        agg = a_ref[0] + a_ref[1]
        h = jnp.maximum(x0 @ w1x_ref[...] + agg @ w1a_ref[...] + b1_ref[...], 0.0)
        xn = x0 + _ln(h @ w2_ref[...] + b2_ref[...], g_ref[...], be_ref[...])
        hm = jnp.maximum(xn @ w1m_ref[...] + b1m_ref[...], 0.0)
        mu = hm @ w2m_ref[...] + b2m_ref[...]
        mu_ref[...] = mu
        hv = jnp.maximum(xn @ w1v_ref[...] + b1v_ref[...], 0.0)
        lv = jnp.clip(hv @ w2v_ref[...] + b2v_ref[...], -10.0, 4.0)
        sig2 = jnp.exp(lv)
        s = jnp.sum(0.5 * (mu * mu + sig2 - lv - 1.0), axis=-1, keepdims=True)
        gidx = i * _BLK_N + lax.broadcasted_iota(jnp.int32, (_BLK_N, 1), 0)
        mask = (gidx < _N).astype(_F32)
        part = jnp.sum(s * mask) * (1.0 / _N)

        @pl.when(i == 0)
        def _():
            kl_ref[...] = jnp.zeros((1, 1), _F32)

        kl_ref[...] = kl_ref[...] + part

    row = pl.BlockSpec((_BLK_N, _H), lambda i: (i, 0))
    return pl.pallas_call(
        body,
        grid=(_NB_N,),
        in_specs=[
            row, pl.BlockSpec((_NC, _BLK_N, _H), lambda i: (0, i, 0)),
            _full((_H, _H)), _full((_H, _H)), _full((1, _H)),
            _full((_H, _H)), _full((1, _H)), _full((1, _H)), _full((1, _H)),
            _full((_H, _H)), _full((1, _H)), _full((_H, 2)), _full((1, 2)),
            _full((_H, _H)), _full((1, _H)), _full((_H, 2)), _full((1, 2)),
        ],
        out_specs=[pl.BlockSpec((_BLK_N, 2), lambda i: (i, 0)),
                   pl.BlockSpec((1, 1), lambda i: (0, 0))],
        out_shape=[jax.ShapeDtypeStruct((_NP, 2), _F32),
                   jax.ShapeDtypeStruct((1, 1), _F32)],
    )(x, agg2, W1x, W1a, b1, W2, b2, g, beta,
      W1m, b1m, W2m, b2m, W1v, b1v, W2v, b2v)


_gather128 = _make_gather_combine(_H)
_scatter = _make_scatter_add()


def _r(v):
    return v.reshape(1, -1)


def kernel(pos, vel_history, particle_type, edge_index, params):
    p = params
    ne, ee = p["node_enc"], p["edge_enc"]
    blk0, blk1 = p["blocks"][0], p["blocks"][1]

    W1n = ne["W1"]
    W1vp = W1n[:12].at[8:10].add(W1n[12:14])
    TE = p["type_embed"] @ W1n[14:30]
    Wxy, w1r2 = ee["W1"][:2], _r(ee["W1"][2])
    gram = Wxy @ Wxy.T
    a, b, c = gram[0, 0], gram[0, 1], gram[1, 1]
    det = a * c - b * b
    gram_inv = jnp.stack([jnp.stack([c, -b]), jnp.stack([-b, a])]) / det
    Pinv = Wxy.T @ gram_inv

    def esplit(b):
        w = b["edge_mlp"]["W1"]
        return w[:_H], w[_H:2 * _H], w[2 * _H:]

    Wd0, Ws0, We0 = esplit(blk0)
    Wd1, Ws1, We1 = esplit(blk1)

    def nsplit(b):
        w = b["node_mlp"]["W1"]
        return w[:_H], w[_H:]

    Wx0, Wa0 = nsplit(blk0)
    Wx1, Wa1 = nsplit(blk1)

    vel_flat = vel_history.reshape(_N, -1)
    F = jnp.zeros((_NP, 12), _F32).at[:_N].set(
        jnp.concatenate([vel_flat, pos], axis=1))
    PT3 = jnp.zeros((_NP,), jnp.int32).at[:_N].set(
        particle_type.astype(jnp.int32)).reshape(_NB_N, 1, _BLK_N)
    src = edge_index[0].astype(jnp.int32)
    dst = edge_index[1].astype(jnp.int32)

    em0, nm0 = blk0["edge_mlp"], blk0["node_mlp"]
    x, Aq, Bq, A1, B1 = _node_enc(
        F, PT3, W1vp, TE, _r(ne["b1"]), ne["W2"], _r(ne["b2"]),
        _r(ne["g"]), _r(ne["beta"]), Wxy, _r(ee["b1"]), Wd0, Ws0,
        _r(blk0["edge_mlp"]["b1"]))
    G0 = _gather128(Aq, Bq, dst, src)
    G1 = _gather128(A1, B1, dst, src)
    e = _edge_block0(G0, G1, Pinv, _r(ee["b1"]), w1r2, ee["W2"], _r(ee["b2"]),
                     _r(ee["g"]), _r(ee["beta"]), We0, em0["W2"],
                     _r(em0["b2"]), _r(em0["g"]), _r(em0["beta"]))
    agg2 = _scatter(e, dst)
    x, A, B = _node_update(
        x, agg2, Wx0, Wa0, _r(nm0["b1"]), nm0["W2"], _r(nm0["b2"]),
        _r(nm0["g"]), _r(nm0["beta"]), Wd1, Ws1, _r(blk1["edge_mlp"]["b1"]))

    em1, nm1 = blk1["edge_mlp"], blk1["node_mlp"]
    G = _gather128(A, B, dst, src)
    e = _edge_block(G, e, We1, em1["W2"], _r(em1["b2"]), _r(em1["g"]),
                    _r(em1["beta"]))
    agg2 = _scatter(e, dst)
    mh, vh = p["mu_head"], p["logv_head"]
    mu, kl = _node_final(
        x, agg2, Wx1, Wa1, _r(nm1["b1"]), nm1["W2"], _r(nm1["b2"]),
        _r(nm1["g"]), _r(nm1["beta"]),
        mh["W1"], _r(mh["b1"]), mh["W2"], _r(mh["b2"]),
        vh["W1"], _r(vh["b1"]), vh["W2"], _r(vh["b2"]))

    return mu[:_N], kl.reshape(())


# speedup vs baseline: 4.3170x; 1.0412x over previous
import functools

import jax
import jax.numpy as jnp
import numpy as np
from jax import lax
from jax.experimental import pallas as pl
from jax.experimental.pallas import tpu as pltpu
from jax.experimental.pallas import tpu_sc as plsc

_N = 10000
_E = 320000
_H = 128
_NP = 10240
_BLK_N = 512
_NB_N = _NP // _BLK_N
_BLK_E = 2000
_NB_E = _E // _BLK_E

_NC = 2
_NS = 16
_NW = _NC * _NS
_EPW = _E // _NW
_C = 80
_NCH = _EPW // _C
_RPT = _NP // _NS

_F32 = jnp.float32


def _sc_mesh():
    return plsc.VectorSubcoreMesh(
        core_axis_name="c", subcore_axis_name="s",
        num_cores=_NC, num_subcores=_NS)


def _make_gather_combine(npair, nbuf):
    @functools.partial(
        pl.kernel,
        out_type=[jax.ShapeDtypeStruct((_E, _H), _F32)] * npair,
        mesh=_sc_mesh(),
        scratch_types=(
            [pltpu.VMEM((_EPW,), jnp.int32)] * 2
            + [pltpu.VMEM((nbuf, _C, _H), _F32)] * (2 * npair)
            + [pltpu.SemaphoreType.DMA] * (3 * npair * nbuf)
        ),
    )
    def k(*refs):
        tabs = refs[:2 * npair]
        ia_hbm, ib_hbm = refs[2 * npair:2 * npair + 2]
        outs = refs[2 * npair + 2:3 * npair + 2]
        scr = refs[3 * npair + 2:]
        ia_v, ib_v = scr[0], scr[1]
        rbufs = scr[2:2 + 2 * npair]
        sems = scr[2 + 2 * npair:]
        sga = sems[:npair * nbuf]
        sgb = sems[npair * nbuf:2 * npair * nbuf]
        swb = sems[2 * npair * nbuf:]

        wid = lax.axis_index("s") * _NC + lax.axis_index("c")
        base = wid * _EPW
        pltpu.sync_copy(ia_hbm.at[pl.ds(base, _EPW)], ia_v)
        pltpu.sync_copy(ib_hbm.at[pl.ds(base, _EPW)], ib_v)

        def g_args(j, b):
            ia_sl = ia_v.at[pl.ds(j * _C, _C)]
            ib_sl = ib_v.at[pl.ds(j * _C, _C)]
            out = []
            for p in range(npair):
                out.append((tabs[2 * p].at[ia_sl], rbufs[2 * p].at[b],
                            sga[p * nbuf + b]))
                out.append((tabs[2 * p + 1].at[ib_sl], rbufs[2 * p + 1].at[b],
                            sgb[p * nbuf + b]))
            return out

        def fire_gather(j, b):
            for args in g_args(j, b):
                pltpu.async_copy(*args)

        def wait_gather(j, b):
            for args in g_args(j, b):
                pltpu.make_async_copy(*args).wait()

        def wb_args(j, b, p):
            return (rbufs[2 * p].at[b], outs[p].at[pl.ds(base + j * _C, _C)],
                    swb[p * nbuf + b])

        def step(j, b):
            @pl.when(j < _NCH)
            def _():
                wait_gather(j, b)

                def row(r, cc):
                    for p in range(npair):
                        for q in range(_H // 16):
                            sl = pl.ds(q * 16, 16)
                            rbufs[2 * p][b, r, sl] = (
                                rbufs[2 * p][b, r, sl]
                                + rbufs[2 * p + 1][b, r, sl])
                    return cc

                lax.fori_loop(0, _C, row, 0)
                for p in range(npair):
                    pltpu.async_copy(*wb_args(j, b, p))

                @pl.when(j + 1 < _NCH)
                def _():
                    b2 = (b + 1) % nbuf

                    @pl.when(j + 1 >= nbuf)
                    def _():
                        for p in range(npair):
                            pltpu.make_async_copy(
                                *wb_args(j + 1 - nbuf, b2, p)).wait()

                    fire_gather(j + 1, b2)

        fire_gather(0, 0)

        def body(jj, carry):
            for t in range(nbuf):
                step(jj * nbuf + t, t)
            return carry

        lax.fori_loop(0, (_NCH + nbuf - 1) // nbuf, body, 0)
        for j in range(_NCH - nbuf, _NCH):
            for p in range(npair):
                pltpu.make_async_copy(*wb_args(j, j % nbuf, p)).wait()

    return k


def _make_scatter_add():
    @functools.partial(
        pl.kernel,
        out_type=jax.ShapeDtypeStruct((_NC, _NP, _H), _F32),
        mesh=_sc_mesh(),
        scratch_types=[
            pltpu.VMEM((2, _C), jnp.int32),
            pltpu.VMEM((2, _C, _H), _F32),
            pltpu.VMEM_SHARED((_NP, _H), _F32),
            pltpu.SemaphoreType.DMA, pltpu.SemaphoreType.DMA,
            pltpu.SemaphoreType.DMA, pltpu.SemaphoreType.DMA,
        ],
    )
    def k(e_hbm, idx_hbm, out_hbm, ib_v, ev, agg_sh, si0, si1, se0, se1):
        si = (si0, si1)
        se = (se0, se1)
        cid = lax.axis_index("c")
        sid = lax.axis_index("s")
        wid = sid * _NC + cid

        def zrow(r, cc):
            for q in range(_H // 16):
                ev[0, r, pl.ds(q * 16, 16)] = jnp.zeros((16,), _F32)
            return cc

        lax.fori_loop(0, _C, zrow, 0)

        def zcp(q, cc):
            pltpu.sync_copy(ev.at[0], agg_sh.at[pl.ds(sid * _RPT + q * _C, _C)])
            return cc

        lax.fori_loop(0, _RPT // _C, zcp, 0)
        plsc.subcore_barrier()

        base = wid * _EPW

        def ld_args(j, b):
            off = base + j * _C
            return ((idx_hbm.at[pl.ds(off, _C)], ib_v.at[b], si[b]),
                    (e_hbm.at[pl.ds(off, _C)], ev.at[b], se[b]))

        def fire_loads(j, b):
            for args in ld_args(j, b):
                pltpu.async_copy(*args)

        def step(j, b):
            @pl.when(j < _NCH)
            def _():
                for args in ld_args(j, b):
                    pltpu.make_async_copy(*args).wait()
                pltpu.sync_copy(ev.at[b], agg_sh.at[ib_v.at[b]], add=True)

                @pl.when(j + 2 < _NCH)
                def _():
                    fire_loads(j + 2, b)

        fire_loads(0, 0)
        fire_loads(1, 1)

        def body(jj, carry):
            step(2 * jj, 0)
            step(2 * jj + 1, 1)
            return carry

        lax.fori_loop(0, (_NCH + 1) // 2, body, 0)
        plsc.subcore_barrier()
        pltpu.sync_copy(agg_sh.at[pl.ds(sid * _RPT, _RPT)],
                        out_hbm.at[cid, pl.ds(sid * _RPT, _RPT)])

    return k


def _ln(o, g, beta):
    mu = jnp.mean(o, axis=-1, keepdims=True)
    var = jnp.mean((o - mu) ** 2, axis=-1, keepdims=True)
    return (o - mu) * lax.rsqrt(var + 1e-5) * g + beta


def _full(shape):
    nd = len(shape)
    return pl.BlockSpec(shape, lambda i: (0,) * nd)


def _node_enc(F, PT3, W1vp, TE, b1n, W2n, b2n, gn, bn, Wxy, b1e, Wd, Ws, b11):
    def body(f_ref, pt_ref, w1_ref, te_ref, b1_ref, w2_ref, b2_ref, g_ref,
             be_ref, wxy_ref, b1e_ref, wd_ref, ws_ref, b11_ref,
             x_ref, aq_ref, bq_ref, a1_ref, b1o_ref):
        f = f_ref[...]
        pt = pt_ref[0, 0, :]
        oh = (pt[:, None] == lax.broadcasted_iota(jnp.int32, (_BLK_N, 8), 1)
              ).astype(_F32)
        x1 = f @ w1_ref[...] + oh @ te_ref[...] + b1_ref[...]
        h = jnp.maximum(x1, 0.0)
        x = _ln(h @ w2_ref[...] + b2_ref[...], g_ref[...], be_ref[...])
        x_ref[...] = x
        q = f[:, 10:12] @ wxy_ref[...]
        aq_ref[...] = q + b1e_ref[...]
        bq_ref[...] = -q
        a1_ref[...] = x @ wd_ref[...] + b11_ref[...]
        b1o_ref[...] = x @ ws_ref[...]

    row = pl.BlockSpec((_BLK_N, _H), lambda i: (i, 0))
    return pl.pallas_call(
        body,
        grid=(_NB_N,),
        in_specs=[
            pl.BlockSpec((_BLK_N, 12), lambda i: (i, 0)),
            pl.BlockSpec((1, 1, _BLK_N), lambda i: (i, 0, 0)),
            _full((12, _H)), _full((8, _H)), _full((1, _H)),
            _full((_H, _H)), _full((1, _H)), _full((1, _H)), _full((1, _H)),
            _full((2, _H)), _full((1, _H)),
            _full((_H, _H)), _full((_H, _H)), _full((1, _H)),
        ],
        out_specs=[row, row, row, row, row],
        out_shape=[jax.ShapeDtypeStruct((_NP, _H), _F32)] * 5,
    )(F, PT3, W1vp, TE, b1n, W2n, b2n, gn, bn, Wxy, b1e, Wd, Ws, b11)


def _edge_block0(G0, G1, Pinv, b1e, w1r2, W2e, b2e, ge, bee, We, W2, b2, g,
                 beta):
    def body(g0_ref, g1_ref, pi_ref, b1_ref, w1_ref, w2e_ref, b2e_ref, ge_ref,
             bee_ref, we_ref, w2_ref, b2_ref, g_ref, be_ref, out_ref):
        g0 = g0_ref[...]
        delta = (g0 - b1_ref[...]) @ pi_ref[...]
        dist = jnp.sqrt(jnp.sum(delta * delta, axis=-1, keepdims=True))
        h0 = jnp.maximum(g0 + dist * w1_ref[...], 0.0)
        e0 = _ln(h0 @ w2e_ref[...] + b2e_ref[...], ge_ref[...], bee_ref[...])
        h = jnp.maximum(g1_ref[...].astype(_F32) + e0 @ we_ref[...], 0.0)
        out_ref[...] = _ln(h @ w2_ref[...] + b2_ref[...], g_ref[...], be_ref[...])

    row = pl.BlockSpec((_BLK_E, _H), lambda i: (i, 0))
    return pl.pallas_call(
        body,
        grid=(_NB_E,),
        in_specs=[
            row, row,
            _full((_H, 2)), _full((1, _H)), _full((1, _H)),
            _full((_H, _H)), _full((1, _H)), _full((1, _H)), _full((1, _H)),
            _full((_H, _H)), _full((_H, _H)),
            _full((1, _H)), _full((1, _H)), _full((1, _H)),
        ],
        out_specs=row,
        out_shape=jax.ShapeDtypeStruct((_E, _H), _F32),
    )(G0, G1, Pinv, b1e, w1r2, W2e, b2e, ge, bee, We, W2, b2, g, beta)


def _edge_block(G, e, W1e, W2, b2, g, beta):
    def body(g_ref, e_ref, w1_ref, w2_ref, b2_ref, g_ln, be_ref, out_ref):
        h = jnp.maximum(g_ref[...].astype(_F32) + e_ref[...] @ w1_ref[...], 0.0)
        out_ref[...] = _ln(h @ w2_ref[...] + b2_ref[...], g_ln[...], be_ref[...])

    row = pl.BlockSpec((_BLK_E, _H), lambda i: (i, 0))
    return pl.pallas_call(
        body,
        grid=(_NB_E,),
        in_specs=[row, row, _full((_H, _H)), _full((_H, _H)),
                  _full((1, _H)), _full((1, _H)), _full((1, _H))],
        out_specs=row,
        out_shape=jax.ShapeDtypeStruct((_E, _H), _F32),
    )(G, e, W1e, W2, b2, g, beta)


def _node_update(x, agg2, W1x, W1a, b1, W2, b2, g, beta, Wd, Ws, b1n):
    def body(x_ref, a_ref, w1x_ref, w1a_ref, b1_ref, w2_ref, b2_ref, g_ref,
             be_ref, wd_ref, ws_ref, b1n_ref, xn_ref, an_ref, bn_ref):
        x0 = x_ref[...]
        agg = a_ref[0] + a_ref[1]
        h = jnp.maximum(x0 @ w1x_ref[...] + agg @ w1a_ref[...] + b1_ref[...], 0.0)
        xn = x0 + _ln(h @ w2_ref[...] + b2_ref[...], g_ref[...], be_ref[...])
        xn_ref[...] = xn
        an_ref[...] = xn @ wd_ref[...] + b1n_ref[...]
        bn_ref[...] = xn @ ws_ref[...]

    row = pl.BlockSpec((_BLK_N, _H), lambda i: (i, 0))
    return pl.pallas_call(
        body,
        grid=(_NB_N,),
        in_specs=[
            row, pl.BlockSpec((_NC, _BLK_N, _H), lambda i: (0, i, 0)),
            _full((_H, _H)), _full((_H, _H)), _full((1, _H)),
            _full((_H, _H)), _full((1, _H)), _full((1, _H)), _full((1, _H)),
            _full((_H, _H)), _full((_H, _H)), _full((1, _H)),
        ],
        out_specs=[row, row, row],
        out_shape=[jax.ShapeDtypeStruct((_NP, _H), _F32)] * 3,
    )(x, agg2, W1x, W1a, b1, W2, b2, g, beta, Wd, Ws, b1n)


def _node_final(x, agg2, W1x, W1a, b1, W2, b2, g, beta,
                W1m, b1m, W2m, b2m, W1v, b1v, W2v, b2v):
    def body(x_ref, a_ref, w1x_ref, w1a_ref, b1_ref, w2_ref, b2_ref, g_ref,
             be_ref, w1m_ref, b1m_ref, w2m_ref, b2m_ref,
             w1v_ref, b1v_ref, w2v_ref, b2v_ref, mu_ref, kl_ref):
        i = pl.program_id(0)
        x0 = x_ref[...]
        agg = a_ref[0] + a_ref[1]
        h = jnp.maximum(x0 @ w1x_ref[...] + agg @ w1a_ref[...] + b1_ref[...], 0.0)
        xn = x0 + _ln(h @ w2_ref[...] + b2_ref[...], g_ref[...], be_ref[...])
        hm = jnp.maximum(xn @ w1m_ref[...] + b1m_ref[...], 0.0)
        mu = hm @ w2m_ref[...] + b2m_ref[...]
        mu_ref[...] = mu
        hv = jnp.maximum(xn @ w1v_ref[...] + b1v_ref[...], 0.0)
        lv = jnp.clip(hv @ w2v_ref[...] + b2v_ref[...], -10.0, 4.0)
        sig2 = jnp.exp(lv)
        s = jnp.sum(0.5 * (mu * mu + sig2 - lv - 1.0), axis=-1, keepdims=True)
        gidx = i * _BLK_N + lax.broadcasted_iota(jnp.int32, (_BLK_N, 1), 0)
        mask = (gidx < _N).astype(_F32)
        part = jnp.sum(s * mask) * (1.0 / _N)

        @pl.when(i == 0)
        def _():
            kl_ref[...] = jnp.zeros((1, 1), _F32)

        kl_ref[...] = kl_ref[...] + part

    row = pl.BlockSpec((_BLK_N, _H), lambda i: (i, 0))
    return pl.pallas_call(
        body,
        grid=(_NB_N,),
        in_specs=[
            row, pl.BlockSpec((_NC, _BLK_N, _H), lambda i: (0, i, 0)),
            _full((_H, _H)), _full((_H, _H)), _full((1, _H)),
            _full((_H, _H)), _full((1, _H)), _full((1, _H)), _full((1, _H)),
            _full((_H, _H)), _full((1, _H)), _full((_H, 2)), _full((1, 2)),
            _full((_H, _H)), _full((1, _H)), _full((_H, 2)), _full((1, 2)),
        ],
        out_specs=[pl.BlockSpec((_BLK_N, 2), lambda i: (i, 0)),
                   pl.BlockSpec((1, 1), lambda i: (0, 0))],
        out_shape=[jax.ShapeDtypeStruct((_NP, 2), _F32),
                   jax.ShapeDtypeStruct((1, 1), _F32)],
    )(x, agg2, W1x, W1a, b1, W2, b2, g, beta,
      W1m, b1m, W2m, b2m, W1v, b1v, W2v, b2v)


_gather1 = _make_gather_combine(1, 3)
_gather2 = _make_gather_combine(2, 2)
_scatter = _make_scatter_add()


def _r(v):
    return v.reshape(1, -1)


def kernel(pos, vel_history, particle_type, edge_index, params):
    p = params
    ne, ee = p["node_enc"], p["edge_enc"]
    blk0, blk1 = p["blocks"][0], p["blocks"][1]

    W1n = ne["W1"]
    W1vp = W1n[:12].at[8:10].add(W1n[12:14])
    TE = p["type_embed"] @ W1n[14:30]
    Wxy, w1r2 = ee["W1"][:2], _r(ee["W1"][2])
    gram = Wxy @ Wxy.T
    a, b, c = gram[0, 0], gram[0, 1], gram[1, 1]
    det = a * c - b * b
    gram_inv = jnp.stack([jnp.stack([c, -b]), jnp.stack([-b, a])]) / det
    Pinv = Wxy.T @ gram_inv

    def esplit(b):
        w = b["edge_mlp"]["W1"]
        return w[:_H], w[_H:2 * _H], w[2 * _H:]

    Wd0, Ws0, We0 = esplit(blk0)
    Wd1, Ws1, We1 = esplit(blk1)

    def nsplit(b):
        w = b["node_mlp"]["W1"]
        return w[:_H], w[_H:]

    Wx0, Wa0 = nsplit(blk0)
    Wx1, Wa1 = nsplit(blk1)

    vel_flat = vel_history.reshape(_N, -1)
    F = jnp.zeros((_NP, 12), _F32).at[:_N].set(
        jnp.concatenate([vel_flat, pos], axis=1))
    PT3 = jnp.zeros((_NP,), jnp.int32).at[:_N].set(
        particle_type.astype(jnp.int32)).reshape(_NB_N, 1, _BLK_N)
    src = edge_index[0].astype(jnp.int32)
    dst = edge_index[1].astype(jnp.int32)

    em0, nm0 = blk0["edge_mlp"], blk0["node_mlp"]
    x, Aq, Bq, A1, B1 = _node_enc(
        F, PT3, W1vp, TE, _r(ne["b1"]), ne["W2"], _r(ne["b2"]),
        _r(ne["g"]), _r(ne["beta"]), Wxy, _r(ee["b1"]), Wd0, Ws0,
        _r(blk0["edge_mlp"]["b1"]))
    G0, G1 = _gather2(Aq, Bq, A1, B1, dst, src)
    e = _edge_block0(G0, G1, Pinv, _r(ee["b1"]), w1r2, ee["W2"], _r(ee["b2"]),
                     _r(ee["g"]), _r(ee["beta"]), We0, em0["W2"],
                     _r(em0["b2"]), _r(em0["g"]), _r(em0["beta"]))
    agg2 = _scatter(e, dst)
    x, A, B = _node_update(
        x, agg2, Wx0, Wa0, _r(nm0["b1"]), nm0["W2"], _r(nm0["b2"]),
        _r(nm0["g"]), _r(nm0["beta"]), Wd1, Ws1, _r(blk1["edge_mlp"]["b1"]))

    em1, nm1 = blk1["edge_mlp"], blk1["node_mlp"]
    G, = _gather1(A, B, dst, src)
    e = _edge_block(G, e, We1, em1["W2"], _r(em1["b2"]),
                    _r(em1["g"]), _r(em1["beta"]))
    agg2 = _scatter(e, dst)
    mh, vh = p["mu_head"], p["logv_head"]
    mu, kl = _node_final(
        x, agg2, Wx1, Wa1, _r(nm1["b1"]), nm1["W2"], _r(nm1["b2"]),
        _r(nm1["g"]), _r(nm1["beta"]),
        mh["W1"], _r(mh["b1"]), mh["W2"], _r(mh["b2"]),
        vh["W1"], _r(vh["b1"]), vh["W2"], _r(vh["b2"]))

    return mu[:_N], kl.reshape(())
